# baseline (speedup 1.0000x reference)
import functools

import jax
import jax.numpy as jnp
from jax import lax
from jax.experimental import pallas as pl
from jax.experimental.pallas import tpu as pltpu
from jax.experimental.pallas import tpu_sc as plsc

N_NODES = 10000
C = 128
CH = 64
N_EDGE = 320000
NBLK = 79
N_PAD = NBLK * C
NSUB = 16
ECH = 160
EPT = ECH * C
E_PAD = NSUB * EPT
STRIPE = N_PAD // NSUB
NBUF = 5
KPRE = 2
DRN = NBUF - KPRE
IDB = 10
IDPRE = IDB - DRN
NEG_SLOPE = 0.2
F32 = jnp.float32


def _leaky(t):
    return jnp.maximum(t, NEG_SLOPE * t)


def _pre_body(x_ref, w_ref, emb_ref, ati_ref, atj_ref, aemi_ref, aemj_ref,
              xh_ref, ai_ref, aj_ref, m_ref):
    i = pl.program_id(0)
    xh = jnp.dot(x_ref[...], w_ref[...], preferred_element_type=F32)
    xh_ref[0, :, :] = xh[:, :CH]
    xh_ref[1, :, :] = xh[:, CH:]
    emb = emb_ref[...]
    ai = (jnp.sum(xh * ati_ref[...], axis=1, keepdims=True)
          + jnp.sum(emb * aemi_ref[...], axis=1, keepdims=True))
    aj = (jnp.sum(xh * atj_ref[...], axis=1, keepdims=True)
          + jnp.sum(emb * aemj_ref[...], axis=1, keepdims=True))
    ai_ref[...] = ai
    aj_ref[...] = aj
    m = jnp.max(aj)

    @pl.when(i == 0)
    def _():
        m_ref[0, 0] = m

    m_ref[0, 0] = jnp.maximum(m_ref[0, 0], m)


_pre_call = pl.pallas_call(
    _pre_body,
    grid=(NBLK,),
    in_specs=[
        pl.BlockSpec((C, C), lambda i: (i, 0)),
        pl.BlockSpec((C, C), lambda i: (0, 0)),
        pl.BlockSpec((C, C), lambda i: (i, 0)),
        pl.BlockSpec((1, C), lambda i: (0, 0)),
        pl.BlockSpec((1, C), lambda i: (0, 0)),
        pl.BlockSpec((1, C), lambda i: (0, 0)),
        pl.BlockSpec((1, C), lambda i: (0, 0)),
    ],
    out_specs=[
        pl.BlockSpec((2, C, CH), lambda i: (0, i, 0)),
        pl.BlockSpec((C, 1), lambda i: (i, 0)),
        pl.BlockSpec((C, 1), lambda i: (i, 0)),
        pl.BlockSpec((1, 1), lambda i: (0, 0), memory_space=pltpu.SMEM),
    ],
    out_shape=[
        jax.ShapeDtypeStruct((2, N_PAD, CH), F32),
        jax.ShapeDtypeStruct((N_PAD, 1), F32),
        jax.ShapeDtypeStruct((N_PAD, 1), F32),
        jax.ShapeDtypeStruct((1, 1), F32),
    ],
)


_mesh = plsc.VectorSubcoreMesh(core_axis_name="c", subcore_axis_name="s")


@functools.partial(
    pl.kernel,
    out_type=(jax.ShapeDtypeStruct((2, N_PAD, CH), F32),
              jax.ShapeDtypeStruct((NSUB, N_PAD), F32)),
    mesh=_mesh,
    compiler_params=pltpu.CompilerParams(needs_layout_passes=False,
                                         use_tc_tiling_on_sc=False),
    scratch_types=[
        pltpu.VMEM((IDB, C), jnp.int32),
        pltpu.VMEM((IDB, C), jnp.int32),
        pltpu.VMEM((N_PAD,), F32),
        pltpu.VMEM((N_PAD,), F32),
        pltpu.VMEM((N_PAD,), F32),
        pltpu.VMEM((NBUF, C, CH), F32),
        pltpu.VMEM((C,), F32),
        pltpu.VMEM((16,), F32),
        pltpu.VMEM_SHARED((N_PAD, CH), F32),
        [pltpu.SemaphoreType.DMA] * IDB,
        [pltpu.SemaphoreType.DMA] * NBUF,
        [pltpu.SemaphoreType.DMA] * NBUF,
    ],
)
def _sc_call(src_hbm, dst_hbm, xh_hbm, ai_hbm, aj_hbm, m_hbm, zn_hbm, zd_hbm,
             num_out, den_out,
             src_v, dst_v, ai_v, aj_v, den_v, rows_v, e1_v, m_v, num_sh,
             isems, gsems, ssems):
    cid = lax.axis_index("c")
    sid = lax.axis_index("s")
    pltpu.sync_copy(ai_hbm, ai_v)
    pltpu.sync_copy(aj_hbm, aj_v)
    pltpu.sync_copy(m_hbm, m_v)
    pltpu.sync_copy(zd_hbm, den_v)
    pltpu.sync_copy(zn_hbm, num_sh.at[pl.ds(sid * STRIPE, STRIPE)])
    plsc.subcore_barrier()
    mvec = m_v[...]

    def _load_ids(t, sl):
        pltpu.async_copy(src_hbm.at[sid].at[t], src_v.at[sl], isems[sl])
        pltpu.async_copy(dst_hbm.at[sid].at[t], dst_v.at[sl], isems[sl])

    def _wait_ids(t, sl):
        pltpu.make_async_copy(src_hbm.at[sid].at[t], src_v.at[sl],
                              isems[sl]).wait()
        pltpu.make_async_copy(dst_hbm.at[sid].at[t], dst_v.at[sl],
                              isems[sl]).wait()

    def _gather(t, b):
        pltpu.async_copy(xh_hbm.at[cid].at[src_v.at[t % IDB]], rows_v.at[b],
                         gsems[b])

    def _gather_wait(t, b):
        pltpu.make_async_copy(xh_hbm.at[cid].at[src_v.at[t % IDB]],
                              rows_v.at[b], gsems[b]).wait()

    def _scatter(t, b):
        pltpu.async_copy(rows_v.at[b], num_sh.at[dst_v.at[t % IDB]],
                         ssems[b], add=True)

    def _scatter_wait(t, b):
        pltpu.make_async_copy(rows_v.at[b], num_sh.at[dst_v.at[t % IDB]],
                              ssems[b]).wait()

    for t in range(IDPRE):
        _load_ids(t, t)
    for t in range(KPRE):
        _wait_ids(t, t)
        _gather(t, t)

    def oct_body(p, carry):
        for b in range(IDB):
            j = p * IDB + b
            bp = (b + KPRE) % NBUF

            @pl.when(j >= DRN)
            def _():
                _scatter_wait(j - DRN, bp)

            @pl.when(j + KPRE < ECH)
            def _():
                _wait_ids(j + KPRE, (b + KPRE) % IDB)
                _gather(j + KPRE, bp)

            @pl.when(j + IDPRE < ECH)
            def _():
                _load_ids(j + IDPRE, (b + IDPRE) % IDB)

            for g in range(8):
                s = src_v[b, pl.ds(g * 16, 16)]
                d = dst_v[b, pl.ds(g * 16, 16)]
                ajs = plsc.load_gather(aj_v, [s])
                aid = plsc.load_gather(ai_v, [d])
                alpha = _leaky(aid + ajs)
                kd = _leaky(aid + mvec)
                e1 = jnp.exp(alpha - kd)
                e1 = jnp.where(s == d, jnp.zeros_like(e1), e1)
                e1_v[pl.ds(g * 16, 16)] = e1
                plsc.addupdate_scatter(den_v, [d], e1)

            _gather_wait(j, b % NBUF)

            @plsc.parallel_loop(0, C, unroll=8)
            def _(e):
                w = plsc.load_gather(e1_v, [jnp.full((16,), e, jnp.int32)])
                for r in range(CH // 16):
                    rows_v[b % NBUF, e, pl.ds(r * 16, 16)] = (
                        rows_v[b % NBUF, e, pl.ds(r * 16, 16)] * w)

            _scatter(j, b % NBUF)
        return carry

    lax.fori_loop(0, ECH // IDB, oct_body, 0)
    for jj in range(ECH - DRN, ECH):
        _scatter_wait(jj, jj % NBUF)
    plsc.subcore_barrier()
    pltpu.sync_copy(num_sh.at[pl.ds(sid * STRIPE, STRIPE)],
                    num_out.at[cid, pl.ds(sid * STRIPE, STRIPE)])

    @pl.when(cid == 0)
    def _():
        pltpu.sync_copy(den_v, den_out.at[sid])


def _post1_body(num_ref, den_ref, ai_ref, aj_ref, m_ref, xh_ref, bias_ref,
                o_ref, ssum_ref, ssq_ref):
    i = pl.program_id(0)
    m = m_ref[0, 0]
    ai = ai_ref[...]
    aj = aj_ref[...]
    kd = _leaky(ai + m)
    e_self = jnp.exp(_leaky(ai + aj) - kd)
    xh = jnp.concatenate([xh_ref[0], xh_ref[1]], axis=-1)
    num = jnp.concatenate([num_ref[0], num_ref[1]], axis=-1) + e_self * xh
    den_lane = jnp.sum(den_ref[...], axis=0)
    den = den_lane.reshape(C, 1) + e_self
    o = num / (den + 1e-16) + bias_ref[...]
    o_ref[...] = o
    row = i * C + lax.broadcasted_iota(jnp.int32, (C, 1), 0)
    om = jnp.where(row < N_NODES, o, jnp.zeros_like(o))

    @pl.when(i == 0)
    def _():
        ssum_ref[...] = jnp.zeros_like(ssum_ref)
        ssq_ref[...] = jnp.zeros_like(ssq_ref)

    ssum_ref[...] += jnp.sum(om, axis=0, keepdims=True)
    ssq_ref[...] += jnp.sum(om * om, axis=0, keepdims=True)


_post1_call = pl.pallas_call(
    _post1_body,
    grid=(NBLK,),
    in_specs=[
        pl.BlockSpec((2, C, CH), lambda i: (0, i, 0)),
        pl.BlockSpec((NSUB, C), lambda i: (0, i)),
        pl.BlockSpec((C, 1), lambda i: (i, 0)),
        pl.BlockSpec((C, 1), lambda i: (i, 0)),
        pl.BlockSpec((1, 1), lambda i: (0, 0), memory_space=pltpu.SMEM),
        pl.BlockSpec((2, C, CH), lambda i: (0, i, 0)),
        pl.BlockSpec((1, C), lambda i: (0, 0)),
    ],
    out_specs=[
        pl.BlockSpec((C, C), lambda i: (i, 0)),
        pl.BlockSpec((1, C), lambda i: (0, 0)),
        pl.BlockSpec((1, C), lambda i: (0, 0)),
    ],
    out_shape=[
        jax.ShapeDtypeStruct((N_PAD, C), F32),
        jax.ShapeDtypeStruct((1, C), F32),
        jax.ShapeDtypeStruct((1, C), F32),
    ],
)


def _post2_body(o_ref, ssum_ref, ssq_ref, g_ref, b_ref, y_ref):
    mean = ssum_ref[...] * (1.0 / N_NODES)
    var = ssq_ref[...] * (1.0 / N_NODES) - mean * mean
    inv = lax.rsqrt(var + 1e-5)
    y = (o_ref[...] - mean) * (inv * g_ref[...]) + b_ref[...]
    y_ref[...] = jnp.maximum(y, 0.0)


_post2_call = pl.pallas_call(
    _post2_body,
    grid=(NBLK,),
    in_specs=[
        pl.BlockSpec((C, C), lambda i: (i, 0)),
        pl.BlockSpec((1, C), lambda i: (0, 0)),
        pl.BlockSpec((1, C), lambda i: (0, 0)),
        pl.BlockSpec((1, C), lambda i: (0, 0)),
        pl.BlockSpec((1, C), lambda i: (0, 0)),
    ],
    out_specs=pl.BlockSpec((C, C), lambda i: (i, 0)),
    out_shape=jax.ShapeDtypeStruct((N_PAD, C), F32),
)


def kernel(x, edge_index, embedding, lin_w, att_i, att_j, att_em_i, att_em_j,
           bias, bn_gamma, bn_beta):
    xp = jnp.zeros((N_PAD, C), F32).at[:N_NODES].set(x)
    ep = jnp.zeros((N_PAD, C), F32).at[:N_NODES].set(embedding)
    ati = att_i.reshape(1, C)
    atj = att_j.reshape(1, C)
    aemi = att_em_i.reshape(1, C)
    aemj = att_em_j.reshape(1, C)
    xh2, ai_col, aj_col, maxaj = _pre_call(xp, lin_w, ep, ati, atj, aemi, aemj)

    src = edge_index[0].astype(jnp.int32)
    dst = edge_index[1].astype(jnp.int32)
    srcp = jnp.zeros((E_PAD,), jnp.int32).at[:N_EDGE].set(src).reshape(NSUB, ECH, C)
    dstp = jnp.zeros((E_PAD,), jnp.int32).at[:N_EDGE].set(dst).reshape(NSUB, ECH, C)
    ai_g = ai_col.reshape(N_PAD)
    aj_g = aj_col.reshape(N_PAD)
    m16 = jnp.broadcast_to(maxaj.reshape(1), (16,))
    zn = jnp.zeros((STRIPE, CH), F32)
    zd = jnp.zeros((N_PAD,), F32)
    num_part, den_part = _sc_call(srcp, dstp, xh2, ai_g, aj_g, m16, zn, zd)

    o, ssum, ssq = _post1_call(num_part, den_part, ai_col, aj_col, maxaj, xh2,
                               bias.reshape(1, C))
    y = _post2_call(o, ssum, ssq, bn_gamma.reshape(1, C), bn_beta.reshape(1, C))
    return y[:N_NODES]


# speedup vs baseline: 1.0083x; 1.0083x over previous
import functools

import jax
import jax.numpy as jnp
from jax import lax
from jax.experimental import pallas as pl
from jax.experimental.pallas import tpu as pltpu
from jax.experimental.pallas import tpu_sc as plsc

N_NODES = 10000
C = 128
CH = 64
N_EDGE = 320000
NBLK = 79
N_PAD = NBLK * C
NSUB = 16
ECH = 160
EPT = ECH * C
E_PAD = NSUB * EPT
STRIPE = N_PAD // NSUB
NBUF = 5
KPRE = 2
DRN = NBUF - KPRE
IDB = 10
IDPRE = IDB - DRN
NEG_SLOPE = 0.2
F32 = jnp.float32


def _leaky(t):
    return jnp.maximum(t, NEG_SLOPE * t)


def _pre_body(x_ref, w_ref, emb_ref, ati_ref, atj_ref, aemi_ref, aemj_ref,
              xh_ref, ai_ref, aj_ref, m_ref):
    i = pl.program_id(0)
    xh = jnp.dot(x_ref[...], w_ref[...], preferred_element_type=F32)
    xh_ref[0, :, :] = xh[:, :CH]
    xh_ref[1, :, :] = xh[:, CH:]
    emb = emb_ref[...]
    ai = (jnp.sum(xh * ati_ref[...], axis=1, keepdims=True)
          + jnp.sum(emb * aemi_ref[...], axis=1, keepdims=True))
    aj = (jnp.sum(xh * atj_ref[...], axis=1, keepdims=True)
          + jnp.sum(emb * aemj_ref[...], axis=1, keepdims=True))
    ai_ref[...] = ai
    aj_ref[...] = aj
    row = i * C + lax.broadcasted_iota(jnp.int32, (C, 1), 0)
    m = jnp.max(jnp.where(row < N_NODES, aj, jnp.full_like(aj, -1e30)))

    @pl.when(i == 0)
    def _():
        m_ref[0, 0] = m

    m_ref[0, 0] = jnp.maximum(m_ref[0, 0], m)


_pre_call = pl.pallas_call(
    _pre_body,
    grid=(NBLK,),
    in_specs=[
        pl.BlockSpec((C, C), lambda i: (i, 0)),
        pl.BlockSpec((C, C), lambda i: (0, 0)),
        pl.BlockSpec((C, C), lambda i: (i, 0)),
        pl.BlockSpec((1, C), lambda i: (0, 0)),
        pl.BlockSpec((1, C), lambda i: (0, 0)),
        pl.BlockSpec((1, C), lambda i: (0, 0)),
        pl.BlockSpec((1, C), lambda i: (0, 0)),
    ],
    out_specs=[
        pl.BlockSpec((2, C, CH), lambda i: (0, i, 0)),
        pl.BlockSpec((C, 1), lambda i: (i, 0)),
        pl.BlockSpec((C, 1), lambda i: (i, 0)),
        pl.BlockSpec((1, 1), lambda i: (0, 0), memory_space=pltpu.SMEM),
    ],
    out_shape=[
        jax.ShapeDtypeStruct((2, N_PAD, CH), F32),
        jax.ShapeDtypeStruct((N_PAD, 1), F32),
        jax.ShapeDtypeStruct((N_PAD, 1), F32),
        jax.ShapeDtypeStruct((1, 1), F32),
    ],
)


_mesh = plsc.VectorSubcoreMesh(core_axis_name="c", subcore_axis_name="s")


@functools.partial(
    pl.kernel,
    out_type=(jax.ShapeDtypeStruct((2, N_PAD, CH), F32),
              jax.ShapeDtypeStruct((NSUB, N_PAD), F32)),
    mesh=_mesh,
    compiler_params=pltpu.CompilerParams(needs_layout_passes=False,
                                         use_tc_tiling_on_sc=False),
    scratch_types=[
        pltpu.VMEM((IDB, C), jnp.int32),
        pltpu.VMEM((IDB, C), jnp.int32),
        pltpu.VMEM((N_PAD,), F32),
        pltpu.VMEM((N_PAD,), F32),
        pltpu.VMEM((N_PAD,), F32),
        pltpu.VMEM((NBUF, C, CH), F32),
        pltpu.VMEM((C,), F32),
        pltpu.VMEM((16,), F32),
        pltpu.VMEM_SHARED((N_PAD, CH), F32),
        [pltpu.SemaphoreType.DMA] * IDB,
        [pltpu.SemaphoreType.DMA] * NBUF,
        [pltpu.SemaphoreType.DMA] * NBUF,
    ],
)
def _sc_call(src_hbm, dst_hbm, xh_hbm, ai_hbm, aj_hbm, m_hbm, zn_hbm, zd_hbm,
             num_out, den_out,
             src_v, dst_v, ai_v, aj_v, den_v, rows_v, e1_v, m_v, num_sh,
             isems, gsems, ssems):
    cid = lax.axis_index("c")
    sid = lax.axis_index("s")
    pltpu.sync_copy(ai_hbm, ai_v)
    pltpu.sync_copy(aj_hbm, aj_v)
    pltpu.sync_copy(m_hbm, m_v)
    pltpu.sync_copy(zd_hbm, den_v)
    pltpu.sync_copy(zn_hbm, num_sh.at[pl.ds(sid * STRIPE, STRIPE)])
    plsc.subcore_barrier()
    mvec = m_v[...]

    def _load_ids(t, sl):
        pltpu.async_copy(src_hbm.at[sid].at[t], src_v.at[sl], isems[sl])
        pltpu.async_copy(dst_hbm.at[sid].at[t], dst_v.at[sl], isems[sl])

    def _wait_ids(t, sl):
        pltpu.make_async_copy(src_hbm.at[sid].at[t], src_v.at[sl],
                              isems[sl]).wait()
        pltpu.make_async_copy(dst_hbm.at[sid].at[t], dst_v.at[sl],
                              isems[sl]).wait()

    def _gather(t, b):
        pltpu.async_copy(xh_hbm.at[cid].at[src_v.at[t % IDB]], rows_v.at[b],
                         gsems[b])

    def _gather_wait(t, b):
        pltpu.make_async_copy(xh_hbm.at[cid].at[src_v.at[t % IDB]],
                              rows_v.at[b], gsems[b]).wait()

    def _scatter(t, b):
        pltpu.async_copy(rows_v.at[b], num_sh.at[dst_v.at[t % IDB]],
                         ssems[b], add=True)

    def _scatter_wait(t, b):
        pltpu.make_async_copy(rows_v.at[b], num_sh.at[dst_v.at[t % IDB]],
                              ssems[b]).wait()

    for t in range(IDPRE):
        _load_ids(t, t)
    for t in range(KPRE):
        _wait_ids(t, t)
        _gather(t, t)

    def oct_body(p, carry):
        for b in range(IDB):
            j = p * IDB + b
            bp = (b + KPRE) % NBUF

            @pl.when(j >= DRN)
            def _():
                _scatter_wait(j - DRN, bp)

            @pl.when(j + KPRE < ECH)
            def _():
                _wait_ids(j + KPRE, (b + KPRE) % IDB)
                _gather(j + KPRE, bp)

            @pl.when(j + IDPRE < ECH)
            def _():
                _load_ids(j + IDPRE, (b + IDPRE) % IDB)

            for g in range(8):
                s = src_v[b, pl.ds(g * 16, 16)]
                d = dst_v[b, pl.ds(g * 16, 16)]
                ajs = plsc.load_gather(aj_v, [s])
                aid = plsc.load_gather(ai_v, [d])
                alpha = _leaky(aid + ajs)
                kd = _leaky(aid + mvec)
                e1 = jnp.exp(alpha - kd)
                e1 = jnp.where(s == d, jnp.zeros_like(e1), e1)
                e1_v[pl.ds(g * 16, 16)] = e1
                plsc.addupdate_scatter(den_v, [d], e1)

            _gather_wait(j, b % NBUF)

            @plsc.parallel_loop(0, C, unroll=8)
            def _(e):
                w = plsc.load_gather(e1_v, [jnp.full((16,), e, jnp.int32)])
                for r in range(CH // 16):
                    rows_v[b % NBUF, e, pl.ds(r * 16, 16)] = (
                        rows_v[b % NBUF, e, pl.ds(r * 16, 16)] * w)

            _scatter(j, b % NBUF)
        return carry

    lax.fori_loop(0, ECH // IDB, oct_body, 0)
    for jj in range(ECH - DRN, ECH):
        _scatter_wait(jj, jj % NBUF)
    plsc.subcore_barrier()
    pltpu.sync_copy(num_sh.at[pl.ds(sid * STRIPE, STRIPE)],
                    num_out.at[cid, pl.ds(sid * STRIPE, STRIPE)])

    @pl.when(cid == 0)
    def _():
        pltpu.sync_copy(den_v, den_out.at[sid])


def _post1_body(num_ref, den_ref, ai_ref, aj_ref, m_ref, xh_ref, bias_ref,
                o_ref, ssum_ref, ssq_ref):
    i = pl.program_id(0)
    m = m_ref[0, 0]
    ai = ai_ref[...]
    aj = aj_ref[...]
    kd = _leaky(ai + m)
    e_self = jnp.exp(_leaky(ai + aj) - kd)
    xh = jnp.concatenate([xh_ref[0], xh_ref[1]], axis=-1)
    num = jnp.concatenate([num_ref[0], num_ref[1]], axis=-1) + e_self * xh
    den_lane = jnp.sum(den_ref[...], axis=0)
    den = den_lane.reshape(C, 1) + e_self
    o = num / (den + 1e-16) + bias_ref[...]
    o_ref[...] = o
    row = i * C + lax.broadcasted_iota(jnp.int32, (C, 1), 0)
    om = jnp.where(row < N_NODES, o, jnp.zeros_like(o))

    @pl.when(i == 0)
    def _():
        ssum_ref[...] = jnp.zeros_like(ssum_ref)
        ssq_ref[...] = jnp.zeros_like(ssq_ref)

    ssum_ref[...] += jnp.sum(om, axis=0, keepdims=True)
    ssq_ref[...] += jnp.sum(om * om, axis=0, keepdims=True)


_post1_call = pl.pallas_call(
    _post1_body,
    grid=(NBLK,),
    in_specs=[
        pl.BlockSpec((2, C, CH), lambda i: (0, i, 0)),
        pl.BlockSpec((NSUB, C), lambda i: (0, i)),
        pl.BlockSpec((C, 1), lambda i: (i, 0)),
        pl.BlockSpec((C, 1), lambda i: (i, 0)),
        pl.BlockSpec((1, 1), lambda i: (0, 0), memory_space=pltpu.SMEM),
        pl.BlockSpec((2, C, CH), lambda i: (0, i, 0)),
        pl.BlockSpec((1, C), lambda i: (0, 0)),
    ],
    out_specs=[
        pl.BlockSpec((C, C), lambda i: (i, 0)),
        pl.BlockSpec((1, C), lambda i: (0, 0)),
        pl.BlockSpec((1, C), lambda i: (0, 0)),
    ],
    out_shape=[
        jax.ShapeDtypeStruct((N_PAD, C), F32),
        jax.ShapeDtypeStruct((1, C), F32),
        jax.ShapeDtypeStruct((1, C), F32),
    ],
)


def _post2_body(o_ref, ssum_ref, ssq_ref, g_ref, b_ref, y_ref):
    mean = ssum_ref[...] * (1.0 / N_NODES)
    var = ssq_ref[...] * (1.0 / N_NODES) - mean * mean
    inv = lax.rsqrt(var + 1e-5)
    y = (o_ref[...] - mean) * (inv * g_ref[...]) + b_ref[...]
    y_ref[...] = jnp.maximum(y, 0.0)


_post2_call = pl.pallas_call(
    _post2_body,
    grid=(NBLK,),
    in_specs=[
        pl.BlockSpec((C, C), lambda i: (i, 0)),
        pl.BlockSpec((1, C), lambda i: (0, 0)),
        pl.BlockSpec((1, C), lambda i: (0, 0)),
        pl.BlockSpec((1, C), lambda i: (0, 0)),
        pl.BlockSpec((1, C), lambda i: (0, 0)),
    ],
    out_specs=pl.BlockSpec((C, C), lambda i: (i, 0)),
    out_shape=jax.ShapeDtypeStruct((N_NODES, C), F32),
)


def kernel(x, edge_index, embedding, lin_w, att_i, att_j, att_em_i, att_em_j,
           bias, bn_gamma, bn_beta):
    ati = att_i.reshape(1, C)
    atj = att_j.reshape(1, C)
    aemi = att_em_i.reshape(1, C)
    aemj = att_em_j.reshape(1, C)
    xp = jnp.zeros((N_PAD, C), F32).at[:N_NODES].set(x)
    ep = jnp.zeros((N_PAD, C), F32).at[:N_NODES].set(embedding)
    xh2, ai_col, aj_col, maxaj = _pre_call(xp, lin_w, ep, ati, atj, aemi, aemj)

    src = edge_index[0].astype(jnp.int32)
    dst = edge_index[1].astype(jnp.int32)
    srcp = jnp.zeros((E_PAD,), jnp.int32).at[:N_EDGE].set(src).reshape(NSUB, ECH, C)
    dstp = jnp.zeros((E_PAD,), jnp.int32).at[:N_EDGE].set(dst).reshape(NSUB, ECH, C)
    ai_g = ai_col.reshape(N_PAD)
    aj_g = aj_col.reshape(N_PAD)
    m16 = jnp.broadcast_to(maxaj.reshape(1), (16,))
    zn = jnp.zeros((STRIPE, CH), F32)
    zd = jnp.zeros((N_PAD,), F32)
    num_part, den_part = _sc_call(srcp, dstp, xh2, ai_g, aj_g, m16, zn, zd)

    o, ssum, ssq = _post1_call(num_part, den_part, ai_col, aj_col, maxaj, xh2,
                               bias.reshape(1, C))
    y = _post2_call(o, ssum, ssq, bn_gamma.reshape(1, C), bn_beta.reshape(1, C))
    return y[:N_NODES]
